# baseline (speedup 1.0000x reference)
import functools
import jax
import jax.numpy as jnp
from jax import lax
from jax.experimental import pallas as pl
from jax.experimental.pallas import tpu as pltpu
from jax.experimental.pallas import tpu_sc as plsc

F32 = jnp.float32
D = 16
H = 33
HP = 48
K = 16
NA, NAP = 5000, 5120
NP, NPP = 12000, 12288
NC, NS = 2, 16
NW = NC * NS


def _leaky(x):
    return jnp.where(x >= 0, x, 0.2 * x)


def _prep_body(x_ref, w1, b1, w2, b2, w1a, b1n, w1b, at_ref, a_ref, g_ref):
    x = x_ref[...]
    h = _leaky(jnp.dot(x, w1[...], preferred_element_type=F32) + b1[...])
    at = jnp.dot(h, w2[...], preferred_element_type=F32) + b2[...]
    at_ref[...] = at
    a_ref[...] = jnp.dot(at, w1a[...], preferred_element_type=F32) + b1n[...]
    g_ref[...] = jnp.dot(at, w1b[...], preferred_element_type=F32)


def _prep_call(atypes_p, w1, b1, w2, b2, w1a, b1n, w1b):
    BL = 512
    n = atypes_p.shape[0]
    row = lambda i: (i, 0)
    whole = lambda i: (0, 0)
    return pl.pallas_call(
        _prep_body,
        grid=(n // BL,),
        in_specs=[pl.BlockSpec((BL, D), row)] + [pl.BlockSpec(w.shape, whole) for w in (w1, b1, w2, b2, w1a, b1n, w1b)],
        out_specs=[pl.BlockSpec((BL, D), row), pl.BlockSpec((BL, HP), row), pl.BlockSpec((BL, HP), row)],
        out_shape=[jax.ShapeDtypeStruct((n, D), F32),
                   jax.ShapeDtypeStruct((n, HP), F32),
                   jax.ShapeDtypeStruct((n, HP), F32)],
    )(atypes_p, w1, b1, w2, b2, w1a, b1n, w1b)


_CHUNK = 1024


def _knn_body(x_ref, yt_ref, idx_ref, *, mask_diag, qb, ncol):
    x = x_ref[...]
    yt = yt_ref[...]
    xn = jnp.sum(x * x, axis=1, keepdims=True)
    yn = jnp.sum(yt * yt, axis=0, keepdims=True)
    xy = jnp.dot(x, yt, preferred_element_type=F32,
                 precision=lax.Precision.HIGHEST)
    d = jnp.maximum(xn + yn - 2.0 * xy, 0.0)
    col = lax.broadcasted_iota(jnp.int32, (qb, ncol), 1)
    if mask_diag:
        row = pl.program_id(0) * qb + lax.broadcasted_iota(jnp.int32, (qb, ncol), 0)
        d = jnp.where(row == col, jnp.inf, d)
    key = (lax.bitcast_convert_type(d, jnp.int32)
           & jnp.int32(0x7FFFFFFF & ~(_CHUNK - 1))) \
        | (col & jnp.int32(_CHUNK - 1))
    nch = ncol // _CHUNK
    big = jnp.int32(0x7FFFFFFF)
    idxs = []
    for _ in range(K):
        b = jnp.min(key[:, 0:_CHUNK], axis=1)
        cid = jnp.zeros((qb,), jnp.int32)
        for c in range(1, nch):
            kc = jnp.min(key[:, c * _CHUNK:(c + 1) * _CHUNK], axis=1)
            better = kc < b
            b = jnp.where(better, kc, b)
            cid = jnp.where(better, jnp.int32(c), cid)
        idxs.append(cid * _CHUNK + (b & jnp.int32(_CHUNK - 1)))
        key = jnp.where(key == b[:, None], big, key)
    idx_ref[...] = jnp.stack(idxs, axis=1)


def _knn_call(x_p, yt, mask_diag):
    QB = 512
    n, ncol = x_p.shape[0], yt.shape[1]
    body = functools.partial(_knn_body, mask_diag=mask_diag, qb=QB, ncol=ncol)
    return pl.pallas_call(
        body,
        grid=(n // QB,),
        in_specs=[pl.BlockSpec((QB, 3), lambda i: (i, 0)),
                  pl.BlockSpec((3, ncol), lambda i: (0, 0))],
        out_specs=pl.BlockSpec((QB, K), lambda i: (i, 0)),
        out_shape=jax.ShapeDtypeStruct((n, K), jnp.int32),
    )(x_p, yt)


def _dist_body(x_ref, e_ref, d_ref, *, qb):
    x0 = x_ref[:, 0]
    x1 = x_ref[:, 1]
    x2 = x_ref[:, 2]
    cols = []
    for j in range(K):
        dx = x0 - e_ref[:, j, 0]
        dy = x1 - e_ref[:, j, 1]
        dz = x2 - e_ref[:, j, 2]
        cols.append(dx * dx + dy * dy + dz * dz)
    d_ref[...] = jnp.stack(cols, axis=1)


def _dist_call(x_p, exyz3):
    QB = 512
    n = x_p.shape[0]
    body = functools.partial(_dist_body, qb=QB)
    return pl.pallas_call(
        body,
        grid=(n // QB,),
        in_specs=[pl.BlockSpec((QB, 3), lambda i: (i, 0)),
                  pl.BlockSpec((QB, K, 16), lambda i: (i, 0, 0))],
        out_specs=pl.BlockSpec((QB, K), lambda i: (i, 0)),
        out_shape=jax.ShapeDtypeStruct((n, K), F32),
    )(x_p, exyz3)


def _make_gather(n_edges, width):
    CH = 128
    SB = 512
    nper = n_edges // NW
    n_super = nper // SB
    assert nper % SB == 0
    mesh = plsc.VectorSubcoreMesh(core_axis_name="c", subcore_axis_name="s")

    @functools.partial(
        pl.kernel,
        mesh=mesh,
        out_type=jax.ShapeDtypeStruct((n_edges, width), F32),
        scratch_types=[pltpu.VMEM((nper,), jnp.int32),
                       pltpu.VMEM((SB, width), F32),
                       pltpu.VMEM((SB, width), F32),
                       pltpu.SemaphoreType.DMA,
                       pltpu.SemaphoreType.DMA,
                       pltpu.SemaphoreType.DMA,
                       pltpu.SemaphoreType.DMA],
        compiler_params=pltpu.CompilerParams(use_tc_tiling_on_sc=False),
    )
    def gather(table_hbm, idx_hbm, out_hbm, idx_all, rows0, rows1,
               gsem0, gsem1, wsem0, wsem1):
        wid = lax.axis_index("s") * NC + lax.axis_index("c")
        base = wid * nper
        pltpu.sync_copy(idx_hbm.at[pl.ds(base, nper)], idx_all)
        rows = (rows0, rows1)
        gsems = (gsem0, gsem1)
        wsems = (wsem0, wsem1)
        writes = [None, None]

        def fire(s):
            buf = rows[s % 2]
            return [
                pltpu.async_copy(
                    table_hbm.at[idx_all.at[pl.ds(s * SB + j * CH, CH)]],
                    buf.at[pl.ds(j * CH, CH)], gsems[s % 2])
                for j in range(SB // CH)
            ]

        pending = fire(0)
        for s in range(n_super):
            nxt = None
            if s + 1 < n_super:
                if writes[(s + 1) % 2] is not None:
                    writes[(s + 1) % 2].wait()
                nxt = fire(s + 1)
            for c in pending:
                c.wait()
            writes[s % 2] = pltpu.async_copy(
                rows[s % 2], out_hbm.at[pl.ds(base + s * SB, SB)], wsems[s % 2])
            pending = nxt
        for w in writes:
            if w is not None:
                w.wait()

    return gather


def _post_body(e_ref, a_ref, d_ref, prev_ref, wd_ref, w2_ref, b2k_ref, gw_ref,
               gb_ref, *proj_and_out, n_proj):
    proj_w = proj_and_out[:2 * n_proj:2]
    proj_b = proj_and_out[1:2 * n_proj:2]
    out_ref = proj_and_out[2 * n_proj]
    proj_refs = proj_and_out[2 * n_proj + 1:]

    a = a_ref[...]
    wd = wd_ref[...]
    s = jnp.zeros(a.shape, F32)
    for j in range(K):
        hj = a + e_ref[:, j, :] + d_ref[:, j][:, None] * wd
        s = s + _leaky(hj)
    msg = jnp.dot(s, w2_ref[...], preferred_element_type=F32) + b2k_ref[...]
    eps = 1e-5
    g0 = msg[:, 0:8]
    g1 = msg[:, 8:16]
    mu0 = jnp.mean(g0, axis=1, keepdims=True)
    mu1 = jnp.mean(g1, axis=1, keepdims=True)
    v0 = jnp.mean((g0 - mu0) ** 2, axis=1, keepdims=True)
    v1 = jnp.mean((g1 - mu1) ** 2, axis=1, keepdims=True)
    xn = jnp.concatenate([(g0 - mu0) / jnp.sqrt(v0 + eps),
                          (g1 - mu1) / jnp.sqrt(v1 + eps)], axis=1)
    out = prev_ref[...] + _leaky(xn * gw_ref[...] + gb_ref[...])
    out_ref[...] = out
    for wref, bref, pref in zip(proj_w, proj_b, proj_refs):
        pref[...] = jnp.dot(out, wref[...], preferred_element_type=F32) + bref[...]


def _post_call(e3, a, dist, prev, wd, w2p, b2k, gw, gb, projs):
    BL = 512
    n = a.shape[0]
    row = lambda i: (i, 0)
    whole = lambda i: (0, 0)
    n_proj = len(projs)
    proj_args = []
    proj_specs = []
    for (w, b) in projs:
        proj_args += [w, b]
        proj_specs += [pl.BlockSpec(w.shape, whole), pl.BlockSpec(b.shape, whole)]
    body = functools.partial(_post_body, n_proj=n_proj)
    out_specs = [pl.BlockSpec((BL, D), row)] + \
                [pl.BlockSpec((BL, w.shape[1]), row) for (w, _) in projs]
    out_shape = [jax.ShapeDtypeStruct((n, D), F32)] + \
                [jax.ShapeDtypeStruct((n, w.shape[1]), F32) for (w, _) in projs]
    res = pl.pallas_call(
        body,
        grid=(n // BL,),
        in_specs=[pl.BlockSpec((BL, K, HP), lambda i: (i, 0, 0)),
                  pl.BlockSpec((BL, HP), row),
                  pl.BlockSpec((BL, K), row),
                  pl.BlockSpec((BL, D), row),
                  pl.BlockSpec(wd.shape, whole),
                  pl.BlockSpec(w2p.shape, whole),
                  pl.BlockSpec(b2k.shape, whole),
                  pl.BlockSpec(gw.shape, whole),
                  pl.BlockSpec(gb.shape, whole)] + proj_specs,
        out_specs=out_specs,
        out_shape=out_shape,
    )(e3, a, dist, prev, wd, w2p, b2k, gw, gb, *proj_args)
    return res


def _pad_h(w):
    return jnp.pad(w, ((0, 0), (0, HP - w.shape[1])))


def kernel(xyz, atom_xyz, atomtypes, batch, atom_batch, tW1, tb1, tW2, tb2,
           aaW1, aab1, aaW2, aab2, aagw, aagb, emW1, emb1, emW2, emb2,
           emgw, emgb):
    atypes_p = jnp.pad(atomtypes, ((0, NAP - NA), (0, 0)))
    ax_p = jnp.pad(atom_xyz, ((0, NAP - NA), (0, 0)), constant_values=1e6)
    x_p = jnp.pad(xyz, ((0, NPP - NP), (0, 0)), constant_values=1e6)
    axT = jnp.transpose(ax_p)

    aaW1a = [_pad_h(aaW1[i][:D, :]) for i in range(3)]
    aaW1b = [_pad_h(aaW1[i][D:2 * D, :]) for i in range(3)]
    aawd = [_pad_h(aaW1[i][2 * D, :][None, :]) for i in range(3)]
    aab1p = [_pad_h(aab1[i][None, :]) for i in range(3)]
    aaW2p = [jnp.pad(aaW2[i], ((0, HP - H), (0, 0))) for i in range(3)]
    aab2k = [K * aab2[i][None, :] for i in range(3)]
    emW1a = [_pad_h(emW1[i][:D, :]) for i in range(3)]
    emW1b = [_pad_h(emW1[i][D:2 * D, :]) for i in range(3)]
    emwd = [_pad_h(emW1[i][2 * D, :][None, :]) for i in range(3)]
    emb1p = [_pad_h(emb1[i][None, :]) for i in range(3)]
    emW2p = [jnp.pad(emW2[i], ((0, HP - H), (0, 0))) for i in range(3)]
    emb2k = [K * emb2[i][None, :] for i in range(3)]
    emW1b_stack = jnp.concatenate(emW1b, axis=1)

    at, A, G = _prep_call(atypes_p, tW1, tb1[None, :], tW2, tb2[None, :],
                          aaW1a[0], aab1p[0], aaW1b[0])

    idxA = _knn_call(ax_p, axT, mask_diag=True)
    idxP = _knn_call(x_p, axT, mask_diag=False)
    idxA_flat = idxA.reshape(-1)
    idxP_flat = idxP.reshape(-1)

    gather_a = _make_gather(NAP * K, HP)
    gather_p = _make_gather(NPP * K, HP)
    gather_ac = _make_gather(NAP * K, 16)
    gather_pc = _make_gather(NPP * K, 16)

    ctab = jnp.pad(ax_p, ((0, 0), (0, 13)))
    eca = gather_ac(ctab, idxA_flat).reshape(NAP, K, 16)
    ecp = gather_pc(ctab, idxP_flat).reshape(NPP, K, 16)
    dA = _dist_call(ax_p, eca)
    dP = _dist_call(x_p, ecp)

    out = at
    gem3 = None
    for i in range(3):
        e = gather_a(G, idxA_flat)
        e3 = e.reshape(NAP, K, HP)
        zb = jnp.zeros((1, HP), F32)
        if i < 2:
            projs = [(aaW1a[i + 1], aab1p[i + 1]), (aaW1b[i + 1], zb)]
            out, A, G = _post_call(e3, A, dA, out, aawd[i], aaW2p[i],
                                   aab2k[i], aagw[i][None, :], aagb[i][None, :], projs)
        else:
            projs = [(emW1b_stack, jnp.zeros((1, 3 * HP), F32))]
            out, gem3 = _post_call(e3, A, dA, out, aawd[i], aaW2p[i],
                                   aab2k[i], aagw[i][None, :], aagb[i][None, :], projs)
    gem = [gem3[:, j * HP:(j + 1) * HP] for j in range(3)]

    emb = jnp.ones((NPP, D), F32)
    a0_row = jnp.sum(emW1a[0], axis=0, keepdims=True) + emb1p[0]
    A = jnp.tile(a0_row, (NPP, 1))
    for i in range(3):
        e = gather_p(gem[i], idxP_flat)
        e3 = e.reshape(NPP, K, HP)
        projs = [(emW1a[i + 1], emb1p[i + 1])] if i < 2 else []
        res = _post_call(e3, A, dP, emb, emwd[i], emW2p[i], emb2k[i],
                         emgw[i][None, :], emgb[i][None, :], projs)
        emb = res[0]
        if i < 2:
            A = res[1]
    return emb[:NP]


# speedup vs baseline: 1.3310x; 1.3310x over previous
import functools
import jax
import jax.numpy as jnp
from jax import lax
from jax.experimental import pallas as pl
from jax.experimental.pallas import tpu as pltpu
from jax.experimental.pallas import tpu_sc as plsc

F32 = jnp.float32
D = 16
H = 33
HP = 48
K = 16
NA, NAP = 5000, 5120
NP, NPP = 12000, 12288
NC, NS = 2, 16
NW = NC * NS


def _leaky(x):
    return jnp.where(x >= 0, x, 0.2 * x)


def _prep_body(x_ref, ax_ref, w1, b1, w2, b2, w1a, b1n, w1b, at_ref, a_ref, g_ref):
    x = x_ref[...]
    h = _leaky(jnp.dot(x, w1[...], preferred_element_type=F32) + b1[...])
    at = jnp.dot(h, w2[...], preferred_element_type=F32) + b2[...]
    at_ref[...] = at
    a_ref[...] = jnp.dot(at, w1a[...], preferred_element_type=F32) + b1n[...]
    g = jnp.dot(at, w1b[...], preferred_element_type=F32)
    pad = jnp.zeros((g.shape[0], HP2 - HP - 3), F32)
    g_ref[...] = jnp.concatenate([g, ax_ref[...], pad], axis=1)


def _prep_call(atypes_p, ax_p, w1, b1, w2, b2, w1a, b1n, w1b):
    BL = 512
    n = atypes_p.shape[0]
    row = lambda i: (i, 0)
    whole = lambda i: (0, 0)
    return pl.pallas_call(
        _prep_body,
        grid=(n // BL,),
        in_specs=[pl.BlockSpec((BL, D), row), pl.BlockSpec((BL, 3), row)]
        + [pl.BlockSpec(w.shape, whole) for w in (w1, b1, w2, b2, w1a, b1n, w1b)],
        out_specs=[pl.BlockSpec((BL, D), row), pl.BlockSpec((BL, HP), row),
                   pl.BlockSpec((BL, HP2), row)],
        out_shape=[jax.ShapeDtypeStruct((n, D), F32),
                   jax.ShapeDtypeStruct((n, HP), F32),
                   jax.ShapeDtypeStruct((n, HP2), F32)],
    )(atypes_p, ax_p, w1, b1, w2, b2, w1a, b1n, w1b)


_CHUNK = 512


def _knn_body(x_ref, yt_ref, idx_ref, *, mask_diag, qb, ncol):
    x = x_ref[...]
    yt = yt_ref[...]
    xn = jnp.sum(x * x, axis=1, keepdims=True)
    yn = jnp.sum(yt * yt, axis=0, keepdims=True)
    xy = jnp.dot(x, yt, preferred_element_type=F32,
                 precision=lax.Precision.HIGHEST)
    d = jnp.maximum(xn + yn - 2.0 * xy, 0.0)
    col = lax.broadcasted_iota(jnp.int32, (qb, ncol), 1)
    if mask_diag:
        row = pl.program_id(0) * qb + lax.broadcasted_iota(jnp.int32, (qb, ncol), 0)
        d = jnp.where(row == col, jnp.inf, d)
    key_i = (lax.bitcast_convert_type(d, jnp.int32)
             & jnp.int32(0x7FFFFFFF & ~(_CHUNK - 1))) \
        | (col & jnp.int32(_CHUNK - 1))
    key = lax.bitcast_convert_type(key_i, F32)
    nch = ncol // _CHUNK
    inf = jnp.float32(jnp.inf)
    bprev = jnp.full((qb,), -1.0, F32)
    idxs = []
    for _ in range(K):
        b = None
        cid = None
        for c in range(nch):
            kc = key[:, c * _CHUNK:(c + 1) * _CHUNK]
            mc = jnp.min(jnp.where(kc > bprev[:, None], kc, inf), axis=1)
            if b is None:
                b, cid = mc, jnp.zeros((qb,), jnp.int32)
            else:
                better = mc < b
                b = jnp.where(better, mc, b)
                cid = jnp.where(better, jnp.int32(c), cid)
        bprev = b
        idxs.append(cid * _CHUNK
                    + (lax.bitcast_convert_type(b, jnp.int32) & jnp.int32(_CHUNK - 1)))
    idx_ref[...] = jnp.stack(idxs, axis=1)


def _knn_call(x_p, yt, mask_diag):
    QB = 512
    n, ncol = x_p.shape[0], yt.shape[1]
    body = functools.partial(_knn_body, mask_diag=mask_diag, qb=QB, ncol=ncol)
    return pl.pallas_call(
        body,
        grid=(n // QB,),
        in_specs=[pl.BlockSpec((QB, 3), lambda i: (i, 0)),
                  pl.BlockSpec((3, ncol), lambda i: (0, 0))],
        out_specs=pl.BlockSpec((QB, K), lambda i: (i, 0)),
        out_shape=jax.ShapeDtypeStruct((n, K), jnp.int32),
    )(x_p, yt)


HP2 = 64


def _make_gather(n_edges, width):
    CH = 128
    SB = 512
    nper = n_edges // NW
    n_super = nper // SB
    assert nper % SB == 0
    mesh = plsc.VectorSubcoreMesh(core_axis_name="c", subcore_axis_name="s")

    @functools.partial(
        pl.kernel,
        mesh=mesh,
        out_type=jax.ShapeDtypeStruct((n_edges, width), F32),
        scratch_types=[pltpu.VMEM((nper,), jnp.int32),
                       pltpu.VMEM((SB, width), F32),
                       pltpu.VMEM((SB, width), F32),
                       pltpu.SemaphoreType.DMA,
                       pltpu.SemaphoreType.DMA,
                       pltpu.SemaphoreType.DMA,
                       pltpu.SemaphoreType.DMA],
        compiler_params=pltpu.CompilerParams(use_tc_tiling_on_sc=False),
    )
    def gather(table_hbm, idx_hbm, out_hbm, idx_all, rows0, rows1,
               gsem0, gsem1, wsem0, wsem1):
        wid = lax.axis_index("s") * NC + lax.axis_index("c")
        base = wid * nper
        pltpu.sync_copy(idx_hbm.at[pl.ds(base, nper)], idx_all)
        rows = (rows0, rows1)
        gsems = (gsem0, gsem1)
        wsems = (wsem0, wsem1)
        writes = [None, None]

        def fire(s):
            buf = rows[s % 2]
            return [
                pltpu.async_copy(
                    table_hbm.at[idx_all.at[pl.ds(s * SB + j * CH, CH)]],
                    buf.at[pl.ds(j * CH, CH)], gsems[s % 2])
                for j in range(SB // CH)
            ]

        pending = fire(0)
        for s in range(n_super):
            nxt = None
            if s + 1 < n_super:
                if writes[(s + 1) % 2] is not None:
                    writes[(s + 1) % 2].wait()
                nxt = fire(s + 1)
            for c in pending:
                c.wait()
            writes[s % 2] = pltpu.async_copy(
                rows[s % 2], out_hbm.at[pl.ds(base + s * SB, SB)], wsems[s % 2])
            pending = nxt
        for w in writes:
            if w is not None:
                w.wait()

    return gather


def _post_body(*refs, n_proj, e_width, compute_dist, append_coords):
    it = iter(refs)
    e_ref = next(it)
    a_ref = next(it)
    x_ref = next(it) if (compute_dist or append_coords) else None
    d_ref = None if compute_dist else next(it)
    prev_ref = next(it)
    wd_ref = next(it)
    w2_ref = next(it)
    b2k_ref = next(it)
    gw_ref = next(it)
    gb_ref = next(it)
    proj_w = [next(it) for _ in range(n_proj)]
    proj_b = [next(it) for _ in range(n_proj)]
    out_ref = next(it)
    proj_refs = [next(it) for _ in range(n_proj)]
    dist_ref = next(it) if compute_dist else None

    a = a_ref[...]
    wd = wd_ref[...]
    if compute_dist:
        x0, x1, x2 = x_ref[:, 0], x_ref[:, 1], x_ref[:, 2]
        dcols = []
        for j in range(K):
            dx = x0 - e_ref[:, j, HP]
            dy = x1 - e_ref[:, j, HP + 1]
            dz = x2 - e_ref[:, j, HP + 2]
            dcols.append(dx * dx + dy * dy + dz * dz)
        dist_ref[...] = jnp.stack(dcols, axis=1)
    s = jnp.zeros(a.shape, F32)
    for j in range(K):
        dj = dcols[j] if compute_dist else d_ref[:, j]
        hj = a + e_ref[:, j, 0:HP] + dj[:, None] * wd
        s = s + _leaky(hj)
    msg = jnp.dot(s, w2_ref[...], preferred_element_type=F32) + b2k_ref[...]
    eps = 1e-5
    g0 = msg[:, 0:8]
    g1 = msg[:, 8:16]
    mu0 = jnp.mean(g0, axis=1, keepdims=True)
    mu1 = jnp.mean(g1, axis=1, keepdims=True)
    v0 = jnp.mean((g0 - mu0) ** 2, axis=1, keepdims=True)
    v1 = jnp.mean((g1 - mu1) ** 2, axis=1, keepdims=True)
    xn = jnp.concatenate([(g0 - mu0) / jnp.sqrt(v0 + eps),
                          (g1 - mu1) / jnp.sqrt(v1 + eps)], axis=1)
    out = prev_ref[...] + _leaky(xn * gw_ref[...] + gb_ref[...])
    out_ref[...] = out
    for p, (wref, bref, pref) in enumerate(zip(proj_w, proj_b, proj_refs)):
        val = jnp.dot(out, wref[...], preferred_element_type=F32) + bref[...]
        if append_coords and p == 0:
            pad = jnp.zeros((val.shape[0], HP2 - HP - 3), F32)
            val = jnp.concatenate([val, x_ref[...], pad], axis=1)
        pref[...] = val


def _post_call(e3, a, dist, prev, wd, w2p, b2k, gw, gb, projs, xyz=None,
               append_coords=False):
    BL = 512
    n = a.shape[0]
    e_width = e3.shape[2]
    compute_dist = dist is None
    row = lambda i: (i, 0)
    whole = lambda i: (0, 0)
    n_proj = len(projs)
    body = functools.partial(_post_body, n_proj=n_proj, e_width=e_width,
                             compute_dist=compute_dist, append_coords=append_coords)
    args = [e3, a]
    in_specs = [pl.BlockSpec((BL, K, e_width), lambda i: (i, 0, 0)),
                pl.BlockSpec((BL, HP), row)]
    if compute_dist or append_coords:
        args.append(xyz)
        in_specs.append(pl.BlockSpec((BL, 3), row))
    if not compute_dist:
        args.append(dist)
        in_specs.append(pl.BlockSpec((BL, K), row))
    args += [prev, wd, w2p, b2k, gw, gb]
    in_specs += [pl.BlockSpec((BL, D), row),
                 pl.BlockSpec(wd.shape, whole),
                 pl.BlockSpec(w2p.shape, whole),
                 pl.BlockSpec(b2k.shape, whole),
                 pl.BlockSpec(gw.shape, whole),
                 pl.BlockSpec(gb.shape, whole)]
    pwidths = []
    for p, (w, b) in enumerate(projs):
        pw = w.shape[1] + (HP2 - HP if (append_coords and p == 0) else 0)
        pwidths.append(pw)
    args += [w for (w, _) in projs] + [b for (_, b) in projs]
    in_specs += [pl.BlockSpec(w.shape, whole) for (w, _) in projs]
    in_specs += [pl.BlockSpec(b.shape, whole) for (_, b) in projs]
    out_specs = [pl.BlockSpec((BL, D), row)] + \
                [pl.BlockSpec((BL, pw), row) for pw in pwidths]
    out_shape = [jax.ShapeDtypeStruct((n, D), F32)] + \
                [jax.ShapeDtypeStruct((n, pw), F32) for pw in pwidths]
    if compute_dist:
        out_specs.append(pl.BlockSpec((BL, K), row))
        out_shape.append(jax.ShapeDtypeStruct((n, K), F32))
    return pl.pallas_call(
        body,
        grid=(n // BL,),
        in_specs=in_specs,
        out_specs=out_specs,
        out_shape=out_shape,
    )(*args)


def _pad_h(w):
    return jnp.pad(w, ((0, 0), (0, HP - w.shape[1])))


def kernel(xyz, atom_xyz, atomtypes, batch, atom_batch, tW1, tb1, tW2, tb2,
           aaW1, aab1, aaW2, aab2, aagw, aagb, emW1, emb1, emW2, emb2,
           emgw, emgb):
    atypes_p = jnp.pad(atomtypes, ((0, NAP - NA), (0, 0)))
    ax_p = jnp.pad(atom_xyz, ((0, NAP - NA), (0, 0)), constant_values=1e6)
    x_p = jnp.pad(xyz, ((0, NPP - NP), (0, 0)), constant_values=1e6)
    axT = jnp.transpose(ax_p)

    aaW1a = [_pad_h(aaW1[i][:D, :]) for i in range(3)]
    aaW1b = [_pad_h(aaW1[i][D:2 * D, :]) for i in range(3)]
    aawd = [_pad_h(aaW1[i][2 * D, :][None, :]) for i in range(3)]
    aab1p = [_pad_h(aab1[i][None, :]) for i in range(3)]
    aaW2p = [jnp.pad(aaW2[i], ((0, HP - H), (0, 0))) for i in range(3)]
    aab2k = [K * aab2[i][None, :] for i in range(3)]
    emW1a = [_pad_h(emW1[i][:D, :]) for i in range(3)]
    emW1b = [_pad_h(emW1[i][D:2 * D, :]) for i in range(3)]
    emwd = [_pad_h(emW1[i][2 * D, :][None, :]) for i in range(3)]
    emb1p = [_pad_h(emb1[i][None, :]) for i in range(3)]
    emW2p = [jnp.pad(emW2[i], ((0, HP - H), (0, 0))) for i in range(3)]
    emb2k = [K * emb2[i][None, :] for i in range(3)]

    at, A, G = _prep_call(atypes_p, ax_p, tW1, tb1[None, :], tW2, tb2[None, :],
                          aaW1a[0], aab1p[0], aaW1b[0])

    idxA = _knn_call(ax_p, axT, mask_diag=True)
    idxP = _knn_call(x_p, axT, mask_diag=False)
    idxA_flat = idxA.reshape(-1)
    idxP_flat = idxP.reshape(-1)

    gather_a0 = _make_gather(NAP * K, HP2)
    gather_a = _make_gather(NAP * K, HP)
    gather_p0 = _make_gather(NPP * K, HP2)
    gather_p = _make_gather(NPP * K, HP)

    out = at
    dA = None
    gem3 = None
    zb = jnp.zeros((1, HP), F32)
    for i in range(3):
        g_fn, w = (gather_a0, HP2) if i == 0 else (gather_a, HP)
        e3 = g_fn(G, idxA_flat).reshape(NAP, K, w)
        if i == 0:
            projs = [(aaW1a[1], aab1p[1]), (aaW1b[1], zb)]
            out, A, G, dA = _post_call(e3, A, None, out, aawd[i], aaW2p[i],
                                       aab2k[i], aagw[i][None, :],
                                       aagb[i][None, :], projs, xyz=ax_p)
        elif i == 1:
            projs = [(aaW1a[2], aab1p[2]), (aaW1b[2], zb)]
            out, A, G = _post_call(e3, A, dA, out, aawd[i], aaW2p[i],
                                   aab2k[i], aagw[i][None, :], aagb[i][None, :],
                                   projs)
        else:
            projs = [(emW1b[0], zb), (emW1b[1], zb), (emW1b[2], zb)]
            out, gem0, gem1, gem2 = _post_call(
                e3, A, dA, out, aawd[i], aaW2p[i], aab2k[i],
                aagw[i][None, :], aagb[i][None, :], projs, xyz=ax_p,
                append_coords=True)
            gem3 = (gem0, gem1, gem2)

    emb = jnp.ones((NPP, D), F32)
    a0_row = jnp.sum(emW1a[0], axis=0, keepdims=True) + emb1p[0]
    A = jnp.tile(a0_row, (NPP, 1))
    dP = None
    for i in range(3):
        g_fn, w = (gather_p0, HP2) if i == 0 else (gather_p, HP)
        e3 = g_fn(gem3[i], idxP_flat).reshape(NPP, K, w)
        if i == 0:
            projs = [(emW1a[1], emb1p[1])]
            emb, A, dP = _post_call(e3, A, None, emb, emwd[i], emW2p[i],
                                    emb2k[i], emgw[i][None, :],
                                    emgb[i][None, :], projs, xyz=x_p)
        else:
            projs = [(emW1a[i + 1], emb1p[i + 1])] if i < 2 else []
            res = _post_call(e3, A, dP, emb, emwd[i], emW2p[i], emb2k[i],
                             emgw[i][None, :], emgb[i][None, :], projs)
            emb = res[0]
            if i < 2:
                A = res[1]
    return emb[:NP]
